# baseline (speedup 1.0000x reference)
import functools
import math

import jax
import jax.numpy as jnp
from jax import lax
from jax.experimental import pallas as pl
from jax.experimental.pallas import tpu as pltpu
from jax.experimental.pallas import tpu_sc as plsc

N = 10000
E = 160000
D = 256
H1 = 512
EPS = 1e-5
CBN = float(1.0 / math.sqrt(1.0 + EPS))

NC = 2
NS = 16
NW = NC * NS

NPAD = 10240
RPT = NPAD // NW
TRASH = RPT + 4
CH = 3200
NCH = E // CH
BK2 = 64
EPW = E // NW

BN = 400
G = N // BN

_f32 = jnp.float32
_i32 = jnp.int32


@functools.cache
def _sc_mesh():
    return plsc.VectorSubcoreMesh(core_axis_name="c", subcore_axis_name="s",
                                  num_cores=NC, num_subcores=NS)


def _deg_body(adj0, adj1, eye, out, acce, accb, ohb, d0b, d1b):
    c = lax.axis_index("c")
    s = lax.axis_index("s")
    w = s * NC + c

    def _z(i, carry):
        acce[pl.ds(i * 16, 16)] = jnp.zeros((16,), _f32)
        accb[pl.ds(i * 16, 16)] = jnp.zeros((16,), _f32)
        return carry

    lax.fori_loop(0, NPAD // 16, _z, 0)

    base = w * EPW
    pltpu.sync_copy(adj0.at[pl.ds(base, EPW)], d0b.at[pl.ds(0, EPW)])
    pltpu.sync_copy(adj1.at[pl.ds(base, EPW)], d1b.at[pl.ds(0, EPW)])
    pltpu.sync_copy(eye, ohb)

    def _k(k, carry):
        d0 = d0b[pl.ds(k, 16)][0]
        g0 = (d0 // 16) * 16
        acce[pl.ds(g0, 16)] = acce[pl.ds(g0, 16)] + ohb[d0 - g0]
        d1 = d1b[pl.ds(k, 16)][0]
        g1 = (d1 // 16) * 16
        accb[pl.ds(g1, 16)] = accb[pl.ds(g1, 16)] + ohb[d1 - g1]
        return carry

    lax.fori_loop(0, EPW, _k, 0)

    pltpu.sync_copy(acce, out.at[pl.ds(w * NPAD, NPAD)])
    pltpu.sync_copy(accb, out.at[pl.ds(NW * NPAD + w * NPAD, NPAD)])


@functools.cache
def _deg_call_fn():
    return pl.kernel(
        _deg_body,
        out_type=jax.ShapeDtypeStruct((2 * NW * NPAD,), _f32),
        mesh=_sc_mesh(),
        scratch_types=[
            pltpu.VMEM((NPAD,), _f32),
            pltpu.VMEM((NPAD,), _f32),
            pltpu.VMEM((16, 16), _f32),
            pltpu.VMEM((EPW + 24,), _i32),
            pltpu.VMEM((EPW + 24,), _i32),
        ],
    )


def _agg_body(tbl, src, dst, out, acc, srcb, dstb, csrc, cdl, shb, pb, dvbuf,
              rowb, sem):
    c = lax.axis_index("c")
    s = lax.axis_index("s")
    w = s * NC + c
    lo = w * RPT

    def _z(i, carry):
        for j in range(D // 16):
            acc[i, pl.ds(j * 16, 16)] = jnp.zeros((16,), _f32)
        return carry

    lax.fori_loop(0, RPT + 8, _z, 0)

    def _zs(i, carry):
        shb[pl.ds(i * 16, 16)] = jnp.zeros((16,), _i32)
        return carry

    lax.fori_loop(0, 12, _zs, 0)

    iot = lax.iota(_i32, 16)

    def _chunk(ch, carry):
        base = ch * CH
        pltpu.sync_copy(src.at[pl.ds(base, CH)], srcb.at[pl.ds(0, CH)])
        pltpu.sync_copy(dst.at[pl.ds(base, CH)], dstb.at[pl.ds(0, CH)])

        def _p1(q, carry):
            for u in range(4):
                g = q * 4 + u
                sb = u * 48
                sl = pl.ds(g * 16, 16)
                d16 = dstb[sl]
                m = (d16 >= lo) & (d16 < lo + RPT)
                dlv = jnp.where(m, d16 - lo, TRASH)
                p = jnp.where(m, jnp.full((16,), 1, _i32),
                              jnp.full((16,), 0, _i32))
                for sh in (1, 2, 4, 8):
                    shb[pl.ds(sb + 16, 16)] = p
                    p = p + shb[pl.ds(sb + 16 - sh, 16)]
                pb[sl] = p
                dvbuf[sl] = dlv
            return carry

        lax.fori_loop(0, CH // 64, _p1, 0)

        def _p2(g, lanes):
            v = pb[pl.ds(g * 16, 16)]
            cntg = v[15]

            @pl.when(cntg > 0)
            def _():
                dv = dvbuf[pl.ds(g * 16, 16)]
                for l in range(16):
                    @pl.when(dv[l] < RPT)
                    def _(l=l):
                        pos = lanes + v[l] - 1
                        csrc[pl.ds(pos, 16)] = srcb[pl.ds(g * 16 + l, 16)]
                        cdl[pl.ds(pos, 16)] = dvbuf[pl.ds(g * 16 + l, 16)]

            return lanes + cntg

        lanes = lax.fori_loop(0, CH // 16, _p2, jnp.int32(0))

        for t in range(BK2 // 16):
            csrc[pl.ds(lanes + t * 16, 16)] = lo + t * 16 + iot
            cdl[pl.ds(lanes + t * 16, 16)] = jnp.full((16,), TRASH, _i32)

        nb = (lanes + BK2 - 1) // BK2

        def _bat(b, carry2):
            pltpu.async_copy(tbl.at[csrc.at[pl.ds(b * BK2, BK2)]],
                             rowb, sem).wait()

            def _k(k, carry3):
                dl = cdl[pl.ds(b * BK2 + k, 16)][0]

                @pl.when(dl < RPT)
                def _():
                    for j in range(D // 16):
                        sl2 = pl.ds(j * 16, 16)
                        acc[dl, sl2] = acc[dl, sl2] + rowb[k, sl2]

                return carry3

            lax.fori_loop(0, BK2, _k, 0)
            return carry2

        lax.fori_loop(0, nb, _bat, 0)
        return carry

    lax.fori_loop(0, NCH, _chunk, 0)

    pltpu.sync_copy(acc.at[pl.ds(0, RPT)], out.at[pl.ds(lo, RPT)])


@functools.cache
def _agg_call_fn():
    return pl.kernel(
        _agg_body,
        out_type=jax.ShapeDtypeStruct((NPAD, D), _f32),
        mesh=_sc_mesh(),
        scratch_types=[
            pltpu.VMEM((RPT + 8, D), _f32),
            pltpu.VMEM((CH + 40,), _i32),
            pltpu.VMEM((CH + 40,), _i32),
            pltpu.VMEM((CH + BK2 + 32,), _i32),
            pltpu.VMEM((CH + BK2 + 32,), _i32),
            pltpu.VMEM((192,), _i32),
            pltpu.VMEM((CH + 40,), _i32),
            pltpu.VMEM((CH + 40,), _i32),
            pltpu.VMEM((BK2, D), _f32),
            pltpu.SemaphoreType.DMA,
        ],
    )


def _degred_body(pe, pb, de, db):
    de[...] = jnp.sum(pe[...], axis=0, keepdims=True)
    db[...] = jnp.sum(pb[...], axis=0, keepdims=True)


def _prep_body(dege, degb, x, dise, disb, xs):
    ve = 1.0 / jnp.sqrt(dege[...] + 1.0)
    dise[...] = ve
    disb[...] = 1.0 / jnp.sqrt(degb[...] + 1.0)
    xs[...] = x[...] * ve


def _l1_body(s1, xs, dise, W1, b1, t_out, sums, sums2):
    i = pl.program_id(0)
    agg = dise[...] * (s1[...] + xs[...])
    t = jnp.dot(agg, W1[...], preferred_element_type=_f32) + b1[...]
    t_out[...] = t

    @pl.when(i == 0)
    def _():
        sums[...] = jnp.zeros_like(sums)
        sums2[...] = jnp.zeros_like(sums2)

    sums[...] += jnp.sum(t, axis=0, keepdims=True)
    sums2[...] += jnp.sum(t * t, axis=0, keepdims=True)


def _gn_body(t, sums, sums2, gnw, gnb, gnm, disb, hta, htb):
    mean = sums[...] * (1.0 / N)
    et2 = sums2[...] * (1.0 / N)
    ms = gnm[...]
    var = et2 - mean * mean * ms * (2.0 - ms)
    outv = t[...] - ms * mean
    h = jnp.maximum(gnw[...] * outv / jnp.sqrt(var + EPS) + gnb[...], 0.0)
    ht = disb[...] * h
    hta[...] = ht[:, :D]
    htb[...] = ht[:, D:]


def _tr_body(xt, out):
    ii = (lax.broadcasted_iota(_i32, (128, 128), 0)
          == lax.broadcasted_iota(_i32, (128, 128), 1)).astype(_f32)
    out[...] = lax.dot_general(xt[...], ii, (((0,), (0,)), ((), ())),
                               preferred_element_type=_f32)


def _dec_body(s2a, s2b, hta, htb, disb, x, xtT,
              Wm, bm, Wd, bd, Ws, bs,
              f1w, f1b, g2, b2, f2w, f2b, g1, bb1, g0, b0,
              xr, zm, zd, zs):
    db = disb[...]
    a2a = db * (s2a[...] + hta[...])
    a2b = db * (s2b[...] + htb[...])
    a2 = jnp.concatenate([a2a, a2b], axis=1)
    vm = jnp.dot(a2, Wm[...], preferred_element_type=_f32) + bm[...]
    vd = jnp.dot(a2, Wd[...], preferred_element_type=_f32) + bd[...]
    vs = jnp.dot(a2, Ws[...], preferred_element_type=_f32) + bs[...]
    zmv = jnp.exp(vm)
    zm[...] = zmv
    zd[...] = 1.0 / (1.0 + jnp.exp(-vd))
    zs[...] = jnp.exp(vs)
    zv = jnp.maximum(
        (jnp.dot(zmv, f1w[...], preferred_element_type=_f32) + f1b[...])
        * (g2[...] * CBN) + b2[...], 0.0)
    xdv = jnp.maximum(
        jnp.dot(zv, f2w[...], preferred_element_type=_f32) + f2b[...], 0.0)
    xr[...] = (xdv + x[...] * (g1[...] * CBN) + bb1[...]
               + xtT[...] * (g0[...] * CBN) + b0[...])


def _row_spec(cols):
    return pl.BlockSpec((BN, cols), lambda i: (i, 0))


def _full_spec(r, c):
    return pl.BlockSpec((r, c), lambda i: (0, 0))


_degred_call = pl.pallas_call(
    _degred_body,
    grid=(NPAD // 1280,),
    in_specs=[pl.BlockSpec((NW, 1280), lambda i: (0, i)),
              pl.BlockSpec((NW, 1280), lambda i: (0, i))],
    out_specs=[pl.BlockSpec((1, 1280), lambda i: (0, i)),
               pl.BlockSpec((1, 1280), lambda i: (0, i))],
    out_shape=[jax.ShapeDtypeStruct((1, NPAD), _f32),
               jax.ShapeDtypeStruct((1, NPAD), _f32)],
)

_prep_call = pl.pallas_call(
    _prep_body,
    grid=(G,),
    in_specs=[_row_spec(1), _row_spec(1), _row_spec(D)],
    out_specs=[_row_spec(1), _row_spec(1), _row_spec(D)],
    out_shape=[jax.ShapeDtypeStruct((N, 1), _f32),
               jax.ShapeDtypeStruct((N, 1), _f32),
               jax.ShapeDtypeStruct((N, D), _f32)],
)

_l1_call = pl.pallas_call(
    _l1_body,
    grid=(G,),
    in_specs=[_row_spec(D), _row_spec(D), _row_spec(1),
              _full_spec(D, H1), _full_spec(1, H1)],
    out_specs=[_row_spec(H1), _full_spec(1, H1), _full_spec(1, H1)],
    out_shape=[jax.ShapeDtypeStruct((N, H1), _f32),
               jax.ShapeDtypeStruct((1, H1), _f32),
               jax.ShapeDtypeStruct((1, H1), _f32)],
)

_gn_call = pl.pallas_call(
    _gn_body,
    grid=(G,),
    in_specs=[_row_spec(H1), _full_spec(1, H1), _full_spec(1, H1),
              _full_spec(1, H1), _full_spec(1, H1), _full_spec(1, H1),
              _row_spec(1)],
    out_specs=[_row_spec(D), _row_spec(D)],
    out_shape=[jax.ShapeDtypeStruct((N, D), _f32),
               jax.ShapeDtypeStruct((N, D), _f32)],
)

_tr_call = pl.pallas_call(
    _tr_body,
    grid=(D // 128,),
    in_specs=[pl.BlockSpec((128, N), lambda j: (j, 0))],
    out_specs=pl.BlockSpec((N, 128), lambda j: (0, j)),
    out_shape=jax.ShapeDtypeStruct((N, D), _f32),
)

_dec_call = pl.pallas_call(
    _dec_body,
    grid=(G,),
    in_specs=[_row_spec(D), _row_spec(D), _row_spec(D), _row_spec(D),
              _row_spec(1), _row_spec(D), _row_spec(D),
              _full_spec(H1, D), _full_spec(1, D),
              _full_spec(H1, D), _full_spec(1, D),
              _full_spec(H1, D), _full_spec(1, D),
              _full_spec(D, D), _full_spec(1, D),
              _full_spec(1, D), _full_spec(1, D),
              _full_spec(D, D), _full_spec(1, D),
              _full_spec(1, D), _full_spec(1, D),
              _row_spec(1), _row_spec(1)],
    out_specs=[_row_spec(D), _row_spec(D), _row_spec(D), _row_spec(D)],
    out_shape=[jax.ShapeDtypeStruct((N, D), _f32),
               jax.ShapeDtypeStruct((N, D), _f32),
               jax.ShapeDtypeStruct((N, D), _f32),
               jax.ShapeDtypeStruct((N, D), _f32)],
)


def kernel(x, adj, x_t, adj_t, W1, b1, gn_w, gn_b, gn_ms, Wm, bm, Wd, bd,
           Ws, bs, fc1_w, fc1_b, bn2_g, bn2_b, fc2_w, fc2_b, bn1_g, bn1_b,
           bn0_g, bn0_b):
    adj0 = adj[0]
    adj1 = adj[1]

    _deg_call = _deg_call_fn()
    _agg_call = _agg_call_fn()

    eye16 = jnp.eye(16, dtype=_f32)
    degflat = _deg_call(adj0, adj1, eye16)
    degr = degflat.reshape(2, NW, NPAD)
    dege_r, degb_r = _degred_call(degr[0], degr[1])
    dise, disb, xs = _prep_call(dege_r.reshape(NPAD, 1),
                                degb_r.reshape(NPAD, 1), x)

    s1 = _agg_call(xs, adj1, adj0)
    t, sums, sums2 = _l1_call(s1, xs, dise, W1, b1.reshape(1, H1))
    hta, htb = _gn_call(t, sums, sums2, gn_w.reshape(1, H1),
                        gn_b.reshape(1, H1), gn_ms.reshape(1, H1), disb)

    s2a = _agg_call(hta, adj0, adj1)
    s2b = _agg_call(htb, adj0, adj1)

    xtT = _tr_call(x_t)
    xr, zm, zd, zs = _dec_call(
        s2a, s2b, hta, htb, disb, x, xtT,
        Wm, bm.reshape(1, D), Wd, bd.reshape(1, D), Ws, bs.reshape(1, D),
        fc1_w, fc1_b.reshape(1, D), bn2_g.reshape(1, D), bn2_b.reshape(1, D),
        fc2_w, fc2_b.reshape(1, D), bn1_g.reshape(1, D), bn1_b.reshape(1, D),
        bn0_g.reshape(N, 1), bn0_b.reshape(N, 1))
    return (xr, zm, zd, zs)


# speedup vs baseline: 1.0236x; 1.0236x over previous
import functools
import math

import jax
import jax.numpy as jnp
from jax import lax
from jax.experimental import pallas as pl
from jax.experimental.pallas import tpu as pltpu
from jax.experimental.pallas import tpu_sc as plsc

N = 10000
E = 160000
D = 256
H1 = 512
EPS = 1e-5
CBN = float(1.0 / math.sqrt(1.0 + EPS))

NC = 2
NS = 16
NW = NC * NS

NPAD = 10240
RPT = NPAD // NW
TRASH = RPT + 4
CH = 3200
NCH = E // CH
BK2 = 64
EPW = E // NW

BN = 400
G = N // BN

_f32 = jnp.float32
_i32 = jnp.int32


@functools.cache
def _sc_mesh():
    return plsc.VectorSubcoreMesh(core_axis_name="c", subcore_axis_name="s",
                                  num_cores=NC, num_subcores=NS)


def _deg_body(adj0, adj1, eye, out, acce, accb, ohb, d0b, d1b):
    c = lax.axis_index("c")
    s = lax.axis_index("s")
    w = s * NC + c

    def _z(i, carry):
        acce[pl.ds(i * 16, 16)] = jnp.zeros((16,), _f32)
        accb[pl.ds(i * 16, 16)] = jnp.zeros((16,), _f32)
        return carry

    lax.fori_loop(0, NPAD // 16, _z, 0)

    base = w * EPW
    pltpu.sync_copy(adj0.at[pl.ds(base, EPW)], d0b.at[pl.ds(0, EPW)])
    pltpu.sync_copy(adj1.at[pl.ds(base, EPW)], d1b.at[pl.ds(0, EPW)])
    pltpu.sync_copy(eye, ohb)

    def _k(k, carry):
        d0 = d0b[pl.ds(k, 16)][0]
        g0 = (d0 // 16) * 16
        acce[pl.ds(g0, 16)] = acce[pl.ds(g0, 16)] + ohb[d0 - g0]
        d1 = d1b[pl.ds(k, 16)][0]
        g1 = (d1 // 16) * 16
        accb[pl.ds(g1, 16)] = accb[pl.ds(g1, 16)] + ohb[d1 - g1]
        return carry

    lax.fori_loop(0, EPW, _k, 0)

    pltpu.sync_copy(acce, out.at[pl.ds(w * NPAD, NPAD)])
    pltpu.sync_copy(accb, out.at[pl.ds(NW * NPAD + w * NPAD, NPAD)])


@functools.cache
def _deg_call_fn():
    return pl.kernel(
        _deg_body,
        out_type=jax.ShapeDtypeStruct((2 * NW * NPAD,), _f32),
        mesh=_sc_mesh(),
        scratch_types=[
            pltpu.VMEM((NPAD,), _f32),
            pltpu.VMEM((NPAD,), _f32),
            pltpu.VMEM((16, 16), _f32),
            pltpu.VMEM((EPW + 24,), _i32),
            pltpu.VMEM((EPW + 24,), _i32),
        ],
    )


def _agg_body(tbl, src, dst, out, acc, srcb, dstb, csrc, cdl, shb, pb, dvbuf,
              rowb, sem):
    c = lax.axis_index("c")
    s = lax.axis_index("s")
    w = s * NC + c
    lo = w * RPT

    def _z(i, carry):
        for j in range(D // 16):
            acc[i, pl.ds(j * 16, 16)] = jnp.zeros((16,), _f32)
        return carry

    lax.fori_loop(0, RPT + 8, _z, 0)

    def _zs(i, carry):
        shb[pl.ds(i * 16, 16)] = jnp.zeros((16,), _i32)
        return carry

    lax.fori_loop(0, 12, _zs, 0)

    iot = lax.iota(_i32, 16)

    def _chunk(ch, carry):
        base = ch * CH
        pltpu.sync_copy(src.at[pl.ds(base, CH)], srcb.at[pl.ds(0, CH)])
        pltpu.sync_copy(dst.at[pl.ds(base, CH)], dstb.at[pl.ds(0, CH)])

        def _grp4(q, lanes):
            for u in range(4):
                g = q * 4 + u
                sb = u * 48
                sl = pl.ds(g * 16, 16)
                d16 = dstb[sl]
                m = (d16 >= lo) & (d16 < lo + RPT)
                dlv = jnp.where(m, d16 - lo, TRASH)
                p = jnp.where(m, jnp.full((16,), 1, _i32),
                              jnp.full((16,), 0, _i32))
                for sh in (1, 2, 4, 8):
                    shb[pl.ds(sb + 16, 16)] = p
                    p = p + shb[pl.ds(sb + 16 - sh, 16)]
                cntg = p[15]
                dvbuf[sl] = dlv

                @pl.when(cntg > 0)
                def _(lanes=lanes, g=g, dlv=dlv, p=p):
                    for l in range(16):
                        @pl.when(dlv[l] < RPT)
                        def _(l=l):
                            pos = lanes + p[l] - 1
                            csrc[pl.ds(pos, 16)] = srcb[pl.ds(g * 16 + l, 16)]
                            cdl[pl.ds(pos, 16)] = dvbuf[pl.ds(g * 16 + l, 16)]

                lanes = lanes + cntg
            return lanes

        lanes = lax.fori_loop(0, CH // 64, _grp4, jnp.int32(0))

        for t in range(BK2 // 16):
            csrc[pl.ds(lanes + t * 16, 16)] = lo + t * 16 + iot
            cdl[pl.ds(lanes + t * 16, 16)] = jnp.full((16,), TRASH, _i32)

        nb = (lanes + BK2 - 1) // BK2

        def _bat(b, carry2):
            pltpu.async_copy(tbl.at[csrc.at[pl.ds(b * BK2, BK2)]],
                             rowb, sem).wait()

            def _k(k, carry3):
                dl = cdl[pl.ds(b * BK2 + k, 16)][0]

                @pl.when(dl < RPT)
                def _():
                    for j in range(D // 16):
                        sl2 = pl.ds(j * 16, 16)
                        acc[dl, sl2] = acc[dl, sl2] + rowb[k, sl2]

                return carry3

            lax.fori_loop(0, BK2, _k, 0)
            return carry2

        lax.fori_loop(0, nb, _bat, 0)
        return carry

    lax.fori_loop(0, NCH, _chunk, 0)

    pltpu.sync_copy(acc.at[pl.ds(0, RPT)], out.at[pl.ds(lo, RPT)])


@functools.cache
def _agg_call_fn():
    return pl.kernel(
        _agg_body,
        out_type=jax.ShapeDtypeStruct((NPAD, D), _f32),
        mesh=_sc_mesh(),
        scratch_types=[
            pltpu.VMEM((RPT + 8, D), _f32),
            pltpu.VMEM((CH + 40,), _i32),
            pltpu.VMEM((CH + 40,), _i32),
            pltpu.VMEM((CH + BK2 + 32,), _i32),
            pltpu.VMEM((CH + BK2 + 32,), _i32),
            pltpu.VMEM((192,), _i32),
            pltpu.VMEM((CH + 40,), _i32),
            pltpu.VMEM((CH + 40,), _i32),
            pltpu.VMEM((BK2, D), _f32),
            pltpu.SemaphoreType.DMA,
        ],
    )


def _degred_body(pe, pb, de, db):
    de[...] = jnp.sum(pe[...], axis=0, keepdims=True)
    db[...] = jnp.sum(pb[...], axis=0, keepdims=True)


def _prep_body(dege, degb, x, dise, disb, xs):
    ve = 1.0 / jnp.sqrt(dege[...] + 1.0)
    dise[...] = ve
    disb[...] = 1.0 / jnp.sqrt(degb[...] + 1.0)
    xs[...] = x[...] * ve


def _l1_body(s1, xs, dise, W1, b1, t_out, sums, sums2):
    i = pl.program_id(0)
    agg = dise[...] * (s1[...] + xs[...])
    t = jnp.dot(agg, W1[...], preferred_element_type=_f32) + b1[...]
    t_out[...] = t

    @pl.when(i == 0)
    def _():
        sums[...] = jnp.zeros_like(sums)
        sums2[...] = jnp.zeros_like(sums2)

    sums[...] += jnp.sum(t, axis=0, keepdims=True)
    sums2[...] += jnp.sum(t * t, axis=0, keepdims=True)


def _gn_body(t, sums, sums2, gnw, gnb, gnm, disb, hta, htb):
    mean = sums[...] * (1.0 / N)
    et2 = sums2[...] * (1.0 / N)
    ms = gnm[...]
    var = et2 - mean * mean * ms * (2.0 - ms)
    outv = t[...] - ms * mean
    h = jnp.maximum(gnw[...] * outv / jnp.sqrt(var + EPS) + gnb[...], 0.0)
    ht = disb[...] * h
    hta[...] = ht[:, :D]
    htb[...] = ht[:, D:]


def _tr_body(xt, out):
    ii = (lax.broadcasted_iota(_i32, (128, 128), 0)
          == lax.broadcasted_iota(_i32, (128, 128), 1)).astype(_f32)
    out[...] = lax.dot_general(xt[...], ii, (((0,), (0,)), ((), ())),
                               preferred_element_type=_f32)


def _dec_body(s2a, s2b, hta, htb, disb, x, xtT,
              Wm, bm, Wd, bd, Ws, bs,
              f1w, f1b, g2, b2, f2w, f2b, g1, bb1, g0, b0,
              xr, zm, zd, zs):
    db = disb[...]
    a2a = db * (s2a[...] + hta[...])
    a2b = db * (s2b[...] + htb[...])
    a2 = jnp.concatenate([a2a, a2b], axis=1)
    vm = jnp.dot(a2, Wm[...], preferred_element_type=_f32) + bm[...]
    vd = jnp.dot(a2, Wd[...], preferred_element_type=_f32) + bd[...]
    vs = jnp.dot(a2, Ws[...], preferred_element_type=_f32) + bs[...]
    zmv = jnp.exp(vm)
    zm[...] = zmv
    zd[...] = 1.0 / (1.0 + jnp.exp(-vd))
    zs[...] = jnp.exp(vs)
    zv = jnp.maximum(
        (jnp.dot(zmv, f1w[...], preferred_element_type=_f32) + f1b[...])
        * (g2[...] * CBN) + b2[...], 0.0)
    xdv = jnp.maximum(
        jnp.dot(zv, f2w[...], preferred_element_type=_f32) + f2b[...], 0.0)
    xr[...] = (xdv + x[...] * (g1[...] * CBN) + bb1[...]
               + xtT[...] * (g0[...] * CBN) + b0[...])


def _row_spec(cols):
    return pl.BlockSpec((BN, cols), lambda i: (i, 0))


def _full_spec(r, c):
    return pl.BlockSpec((r, c), lambda i: (0, 0))


_degred_call = pl.pallas_call(
    _degred_body,
    grid=(NPAD // 1280,),
    in_specs=[pl.BlockSpec((NW, 1280), lambda i: (0, i)),
              pl.BlockSpec((NW, 1280), lambda i: (0, i))],
    out_specs=[pl.BlockSpec((1, 1280), lambda i: (0, i)),
               pl.BlockSpec((1, 1280), lambda i: (0, i))],
    out_shape=[jax.ShapeDtypeStruct((1, NPAD), _f32),
               jax.ShapeDtypeStruct((1, NPAD), _f32)],
)

_prep_call = pl.pallas_call(
    _prep_body,
    grid=(G,),
    in_specs=[_row_spec(1), _row_spec(1), _row_spec(D)],
    out_specs=[_row_spec(1), _row_spec(1), _row_spec(D)],
    out_shape=[jax.ShapeDtypeStruct((N, 1), _f32),
               jax.ShapeDtypeStruct((N, 1), _f32),
               jax.ShapeDtypeStruct((N, D), _f32)],
)

_l1_call = pl.pallas_call(
    _l1_body,
    grid=(G,),
    in_specs=[_row_spec(D), _row_spec(D), _row_spec(1),
              _full_spec(D, H1), _full_spec(1, H1)],
    out_specs=[_row_spec(H1), _full_spec(1, H1), _full_spec(1, H1)],
    out_shape=[jax.ShapeDtypeStruct((N, H1), _f32),
               jax.ShapeDtypeStruct((1, H1), _f32),
               jax.ShapeDtypeStruct((1, H1), _f32)],
)

_gn_call = pl.pallas_call(
    _gn_body,
    grid=(G,),
    in_specs=[_row_spec(H1), _full_spec(1, H1), _full_spec(1, H1),
              _full_spec(1, H1), _full_spec(1, H1), _full_spec(1, H1),
              _row_spec(1)],
    out_specs=[_row_spec(D), _row_spec(D)],
    out_shape=[jax.ShapeDtypeStruct((N, D), _f32),
               jax.ShapeDtypeStruct((N, D), _f32)],
)

_tr_call = pl.pallas_call(
    _tr_body,
    grid=(D // 128,),
    in_specs=[pl.BlockSpec((128, N), lambda j: (j, 0))],
    out_specs=pl.BlockSpec((N, 128), lambda j: (0, j)),
    out_shape=jax.ShapeDtypeStruct((N, D), _f32),
)

_dec_call = pl.pallas_call(
    _dec_body,
    grid=(G,),
    in_specs=[_row_spec(D), _row_spec(D), _row_spec(D), _row_spec(D),
              _row_spec(1), _row_spec(D), _row_spec(D),
              _full_spec(H1, D), _full_spec(1, D),
              _full_spec(H1, D), _full_spec(1, D),
              _full_spec(H1, D), _full_spec(1, D),
              _full_spec(D, D), _full_spec(1, D),
              _full_spec(1, D), _full_spec(1, D),
              _full_spec(D, D), _full_spec(1, D),
              _full_spec(1, D), _full_spec(1, D),
              _row_spec(1), _row_spec(1)],
    out_specs=[_row_spec(D), _row_spec(D), _row_spec(D), _row_spec(D)],
    out_shape=[jax.ShapeDtypeStruct((N, D), _f32),
               jax.ShapeDtypeStruct((N, D), _f32),
               jax.ShapeDtypeStruct((N, D), _f32),
               jax.ShapeDtypeStruct((N, D), _f32)],
)


def kernel(x, adj, x_t, adj_t, W1, b1, gn_w, gn_b, gn_ms, Wm, bm, Wd, bd,
           Ws, bs, fc1_w, fc1_b, bn2_g, bn2_b, fc2_w, fc2_b, bn1_g, bn1_b,
           bn0_g, bn0_b):
    adj0 = adj[0]
    adj1 = adj[1]

    _deg_call = _deg_call_fn()
    _agg_call = _agg_call_fn()

    eye16 = jnp.eye(16, dtype=_f32)
    degflat = _deg_call(adj0, adj1, eye16)
    degr = degflat.reshape(2, NW, NPAD)
    dege_r, degb_r = _degred_call(degr[0], degr[1])
    dise, disb, xs = _prep_call(dege_r.reshape(NPAD, 1),
                                degb_r.reshape(NPAD, 1), x)

    s1 = _agg_call(xs, adj1, adj0)
    t, sums, sums2 = _l1_call(s1, xs, dise, W1, b1.reshape(1, H1))
    hta, htb = _gn_call(t, sums, sums2, gn_w.reshape(1, H1),
                        gn_b.reshape(1, H1), gn_ms.reshape(1, H1), disb)

    s2a = _agg_call(hta, adj0, adj1)
    s2b = _agg_call(htb, adj0, adj1)

    xtT = _tr_call(x_t)
    xr, zm, zd, zs = _dec_call(
        s2a, s2b, hta, htb, disb, x, xtT,
        Wm, bm.reshape(1, D), Wd, bd.reshape(1, D), Ws, bs.reshape(1, D),
        fc1_w, fc1_b.reshape(1, D), bn2_g.reshape(1, D), bn2_b.reshape(1, D),
        fc2_w, fc2_b.reshape(1, D), bn1_g.reshape(1, D), bn1_b.reshape(1, D),
        bn0_g.reshape(N, 1), bn0_b.reshape(N, 1))
    return (xr, zm, zd, zs)
